# baseline (device time: 7875 ns/iter reference)
import jax
import jax.numpy as jnp
from jax import lax
from jax.experimental import pallas as pl
from jax.experimental.pallas import tpu as pltpu

N_CHUNK = 4


def kernel(x):
    m_per, n_per = x.shape
    rows = m_per // N_CHUNK

    def body(x_ref, out_ref, vbuf, acc_ref, comm_ref,
             load_sems, send_sem, recv_sem, out_sem):
        my_x = lax.axis_index("x")
        my_y = lax.axis_index("y")
        peer = (1 - my_x, my_y)

        barrier_sem = pltpu.get_barrier_semaphore()
        pl.semaphore_signal(
            barrier_sem, inc=1, device_id=peer,
            device_id_type=pl.DeviceIdType.MESH,
        )

        loads = []
        for c in range(N_CHUNK):
            cp = pltpu.make_async_copy(
                x_ref.at[pl.ds(c * rows, rows), :], vbuf.at[c], load_sems.at[c]
            )
            cp.start()
            loads.append(cp)

        loads[0].wait()
        acc_ref[0, :] = jnp.sum(vbuf[0], axis=0)
        for c in range(1, N_CHUNK):
            loads[c].wait()
            acc_ref[0, :] = acc_ref[0, :] + jnp.sum(vbuf[c], axis=0)

        pl.semaphore_wait(barrier_sem, 1)

        rdma = pltpu.make_async_remote_copy(
            src_ref=acc_ref,
            dst_ref=comm_ref,
            send_sem=send_sem,
            recv_sem=recv_sem,
            device_id=peer,
            device_id_type=pl.DeviceIdType.MESH,
        )
        rdma.start()
        rdma.wait()

        acc_ref[0, :] = acc_ref[0, :] + comm_ref[0, :]
        out_cp = pltpu.make_async_copy(acc_ref, out_ref, out_sem)
        out_cp.start()
        out_cp.wait()

    return pl.pallas_call(
        body,
        out_shape=jax.ShapeDtypeStruct((1, n_per), jnp.float32),
        in_specs=[pl.BlockSpec(memory_space=pl.ANY)],
        out_specs=pl.BlockSpec(memory_space=pl.ANY),
        scratch_shapes=[
            pltpu.VMEM((N_CHUNK, rows, n_per), jnp.float32),
            pltpu.VMEM((1, n_per), jnp.float32),
            pltpu.VMEM((1, n_per), jnp.float32),
            pltpu.SemaphoreType.DMA((N_CHUNK,)),
            pltpu.SemaphoreType.DMA,
            pltpu.SemaphoreType.DMA,
            pltpu.SemaphoreType.DMA,
        ],
        compiler_params=pltpu.CompilerParams(collective_id=0),
    )(x)
